# baseline (speedup 1.0000x reference)
import dataclasses
import functools

import jax
import jax.numpy as jnp
from jax import lax
from jax.experimental import pallas as pl
from jax.experimental.pallas import tpu as pltpu
from jax.experimental.pallas import tpu_sc as plsc

DIM = 128
K = 262144
B = 4096
N_NEG = 512
T = 0.09

NC = 2
NS = 16
NW = NC * NS
QPW = B // NW
BK = 4096
NBLK = K // BK
IDX_CHUNK = 128
NCHUNK = N_NEG // IDX_CHUNK


def _prologue_body(q_ref, k_ref, qn_ref, kn_ref, lpos_ref):
    qv = q_ref[...]
    kv = k_ref[...]
    qn = qv / jnp.maximum(jnp.sqrt(jnp.sum(qv * qv, axis=1, keepdims=True)), 1e-12)
    kn = kv / jnp.maximum(jnp.sqrt(jnp.sum(kv * kv, axis=1, keepdims=True)), 1e-12)
    qn_ref[...] = qn
    kn_ref[...] = kn
    lpos_ref[...] = jnp.sum(qn * kn, axis=1, keepdims=True)


def _prologue(q, k):
    return pl.pallas_call(
        _prologue_body,
        out_shape=(
            jax.ShapeDtypeStruct((B, DIM), jnp.float32),
            jax.ShapeDtypeStruct((B, DIM), jnp.float32),
            jax.ShapeDtypeStruct((B, 1), jnp.float32),
        ),
    )(q, k)


def _table_body(ws_ref, pw_ref, starts_ref, tbl_ref, kn_ref, out_ref, bf_ref):
    i = pl.program_id(0)
    out_ref[...] = tbl_ref[...]
    s = starts_ref[i]
    e = starts_ref[i + 1]

    def wr(j, carry):
        row = ws_ref[j] - i * BK
        src = pw_ref[j]
        out_ref[pl.ds(row, 1), :] = kn_ref[pl.ds(src, 1), :]
        return carry

    lax.fori_loop(s, e, wr, 0)
    bf_ref[...] = out_ref[...].astype(jnp.bfloat16)


def _table_scatter(table0, kn, ws, pw, starts):
    return pl.pallas_call(
        _table_body,
        grid=(NBLK,),
        in_specs=[
            pl.BlockSpec(memory_space=pltpu.SMEM),
            pl.BlockSpec(memory_space=pltpu.SMEM),
            pl.BlockSpec(memory_space=pltpu.SMEM),
            pl.BlockSpec((BK, DIM), lambda i: (i, 0)),
            pl.BlockSpec((B, DIM), lambda i: (0, 0)),
        ],
        out_specs=(
            pl.BlockSpec((BK, DIM), lambda i: (i, 0)),
            pl.BlockSpec((BK, DIM), lambda i: (i, 0)),
        ),
        out_shape=(
            jax.ShapeDtypeStruct((K, DIM), jnp.float32),
            jax.ShapeDtypeStruct((K, DIM), jnp.bfloat16),
        ),
    )(ws, pw, starts, table0, kn)


def _untranspose_body(tbl_ref, out_ref):
    out_ref[...] = jnp.transpose(tbl_ref[...])


def _untranspose(table_new):
    return pl.pallas_call(
        _untranspose_body,
        grid=(NBLK,),
        in_specs=[pl.BlockSpec((BK, DIM), lambda i: (i, 0))],
        out_specs=pl.BlockSpec((DIM, BK), lambda i: (0, i)),
        out_shape=jax.ShapeDtypeStruct((DIM, K), jnp.float32),
    )(table_new)


DIMW = DIM // 2


def _lneg_sc_body(table_hbm, qn_hbm, nidx_hbm, out_hbm,
                  idx_v, rows_v, qn_v, out_v,
                  gsem0, gsem1, osem0, osem1):
    wid = lax.axis_index("s") * NC + lax.axis_index("c")
    base = wid * QPW
    gsems = (gsem0, gsem1)
    osems = (osem0, osem1)

    def fire(slot, qi):
        qrow = base + qi
        pltpu.sync_copy(nidx_hbm.at[qrow], idx_v.at[slot])
        pltpu.sync_copy(qn_hbm.at[qrow], qn_v.at[slot])
        for ck in range(NCHUNK):
            pltpu.async_copy(
                table_hbm.at[idx_v.at[slot, ck]],
                rows_v.at[slot, pl.ds(ck * IDX_CHUNK, IDX_CHUNK)],
                gsems[slot],
            )

    def wait_gather(slot):
        for ck in range(NCHUNK):
            pltpu.make_async_copy(
                table_hbm.at[idx_v.at[slot, ck]],
                rows_v.at[slot, pl.ds(ck * IDX_CHUNK, IDX_CHUNK)],
                gsems[slot],
            ).wait()

    lane = lax.broadcasted_iota(jnp.int32, (16,), 0)
    lane15 = jnp.full((16,), 15, jnp.int32)

    def compute(slot, qi):
        qrow = base + qi
        qvecs = [qn_v[slot, pl.ds(t * 16, 16)] for t in range(DIM // 16)]

        @pl.loop(0, N_NEG, step=16)
        def _j_loop(j0):
            res = jnp.zeros((16,), jnp.float32)
            for l in range(16):
                j = j0 + l
                acc = None
                for t in range(DIMW // 16):
                    w = rows_v[slot, j, pl.ds(t * 16, 16)]
                    bf = plsc.bitcast(w, jnp.bfloat16)
                    a, b = plsc.unpack(bf, format=plsc.PackFormat.INTERLEAVED)
                    term = a * qvecs[2 * t] + b * qvecs[2 * t + 1]
                    acc = term if acc is None else acc + term
                tot = jnp.cumsum(acc).at[lane15].get(mode="promise_in_bounds")
                res = jnp.where(lane == l, tot, res)
            out_v[slot, pl.ds(j0, 16)] = res

        pltpu.async_copy(out_v.at[slot], out_hbm.at[qrow], osems[slot])

    fire(0, 0)
    fire(1, 1)

    @pl.loop(0, QPW, step=2)
    def _q_loop(q0):
        for s in range(2):
            qi = q0 + s
            wait_gather(s)

            @pl.when(qi >= 2)
            def _drain_out():
                pltpu.make_async_copy(
                    out_v.at[s], out_hbm.at[base + qi], osems[s]).wait()

            compute(s, qi)

            @pl.when(qi + 2 < QPW)
            def _prefetch():
                fire(s, qi + 2)

    for s in range(2):
        pltpu.make_async_copy(
            out_v.at[s], out_hbm.at[base], osems[s]).wait()


def _lneg_sc(table_i32, qn_perm, nidx3):
    mesh = plsc.VectorSubcoreMesh(core_axis_name="c", subcore_axis_name="s")
    cp = pltpu.CompilerParams()
    if "needs_layout_passes" in pltpu.CompilerParams.__dataclass_fields__:
        cp = dataclasses.replace(cp, needs_layout_passes=False)
    if "use_tc_tiling_on_sc" in pltpu.CompilerParams.__dataclass_fields__:
        cp = dataclasses.replace(cp, use_tc_tiling_on_sc=False)
    kern = pl.kernel(
        _lneg_sc_body,
        out_type=jax.ShapeDtypeStruct((B, N_NEG), jnp.float32),
        mesh=mesh,
        compiler_params=cp,
        scratch_types=[
            pltpu.VMEM((2, NCHUNK, IDX_CHUNK), jnp.int32),
            pltpu.VMEM((2, N_NEG, DIMW), jnp.int32),
            pltpu.VMEM((2, DIM), jnp.float32),
            pltpu.VMEM((2, N_NEG), jnp.float32),
            pltpu.SemaphoreType.DMA,
            pltpu.SemaphoreType.DMA,
            pltpu.SemaphoreType.DMA,
            pltpu.SemaphoreType.DMA,
        ],
    )
    return kern(table_i32, qn_perm, nidx3)


def _epilogue_body(lpos_ref, lneg_ref, out_ref):
    inv_t = jnp.float32(1.0 / T)
    out_ref[...] = jnp.concatenate(
        [lpos_ref[...] * inv_t, lneg_ref[...] * inv_t], axis=1)


def _epilogue(lpos, lneg):
    return pl.pallas_call(
        _epilogue_body,
        out_shape=jax.ShapeDtypeStruct((B, 1 + N_NEG), jnp.float32),
    )(lpos, lneg)


def kernel(q, k, queue, write_idx, neg_idx):
    perm = jnp.argsort(write_idx, stable=True)
    ws = write_idx[perm]
    jstar = jnp.searchsorted(ws, ws, side="right") - 1
    pw = perm[jstar].astype(jnp.int32)
    starts = jnp.searchsorted(
        ws, jnp.arange(NBLK + 1, dtype=jnp.int32) * BK).astype(jnp.int32)
    ws = ws.astype(jnp.int32)

    nidx3 = neg_idx.reshape(B, NCHUNK, IDX_CHUNK)

    qn, kn, lpos = _prologue(q, k)

    table0 = jnp.transpose(queue)
    table_new, table_bf = _table_scatter(table0, kn, ws, pw, starts)
    table_i32 = lax.bitcast_convert_type(
        table_bf.reshape(K, DIMW, 2), jnp.int32)
    qn_perm = qn.reshape(B, 4, 16, 2).transpose(0, 1, 3, 2).reshape(B, DIM)
    lneg = _lneg_sc(table_i32, qn_perm, nidx3)
    queue_new = _untranspose(table_new)
    logits = _epilogue(lpos, lneg)
    labels = jnp.zeros((B,), dtype=jnp.int32)
    return logits, queue_new, labels


# speedup vs baseline: 1.4246x; 1.4246x over previous
import dataclasses
import functools

import jax
import jax.numpy as jnp
from jax import lax
from jax.experimental import pallas as pl
from jax.experimental.pallas import tpu as pltpu
from jax.experimental.pallas import tpu_sc as plsc

DIM = 128
K = 262144
B = 4096
N_NEG = 512
T = 0.09

NC = 2
NS = 16
NW = NC * NS
QPW = B // NW
BK = 4096
NBLK = K // BK
IDX_CHUNK = 128
NCHUNK = N_NEG // IDX_CHUNK


def _prologue_body(q_ref, k_ref, qn_ref, kn_ref, lpos_ref):
    qv = q_ref[...]
    kv = k_ref[...]
    qn = qv / jnp.maximum(jnp.sqrt(jnp.sum(qv * qv, axis=1, keepdims=True)), 1e-12)
    kn = kv / jnp.maximum(jnp.sqrt(jnp.sum(kv * kv, axis=1, keepdims=True)), 1e-12)
    qn_ref[...] = qn
    kn_ref[...] = kn
    lpos_ref[...] = jnp.sum(qn * kn, axis=1, keepdims=True)


def _prologue(q, k):
    return pl.pallas_call(
        _prologue_body,
        out_shape=(
            jax.ShapeDtypeStruct((B, DIM), jnp.float32),
            jax.ShapeDtypeStruct((B, DIM), jnp.float32),
            jax.ShapeDtypeStruct((B, 1), jnp.float32),
        ),
    )(q, k)


def _table_body(ws_ref, pw_ref, starts_ref, tbl_ref, kn_ref, out_ref):
    i = pl.program_id(0)
    out_ref[...] = tbl_ref[...]
    s = starts_ref[i]
    e = starts_ref[i + 1]

    def wr(j, carry):
        row = ws_ref[j] - i * BK
        src = pw_ref[j]
        out_ref[pl.ds(row, 1), :] = kn_ref[pl.ds(src, 1), :]
        return carry

    lax.fori_loop(s, e, wr, 0)


def _table_scatter(table0, kn, ws, pw, starts):
    return pl.pallas_call(
        _table_body,
        grid=(NBLK,),
        in_specs=[
            pl.BlockSpec(memory_space=pltpu.SMEM),
            pl.BlockSpec(memory_space=pltpu.SMEM),
            pl.BlockSpec(memory_space=pltpu.SMEM),
            pl.BlockSpec((BK, DIM), lambda i: (i, 0)),
            pl.BlockSpec((B, DIM), lambda i: (0, 0)),
        ],
        out_specs=pl.BlockSpec((BK, DIM), lambda i: (i, 0)),
        out_shape=jax.ShapeDtypeStruct((K, DIM), jnp.float32),
    )(ws, pw, starts, table0, kn)


def _untranspose_body(tbl_ref, out_ref):
    out_ref[...] = jnp.transpose(tbl_ref[...])


def _untranspose(table_new):
    return pl.pallas_call(
        _untranspose_body,
        grid=(NBLK,),
        in_specs=[pl.BlockSpec((BK, DIM), lambda i: (i, 0))],
        out_specs=pl.BlockSpec((DIM, BK), lambda i: (0, i)),
        out_shape=jax.ShapeDtypeStruct((DIM, K), jnp.float32),
    )(table_new)


NSLOT = 4
NCH = QPW * NCHUNK


def _lneg_sc_body(table_hbm, qn_hbm, nidx_hbm, out_hbm,
                  idx_v, rows_v, qn_v, out_v,
                  gsem0, gsem1, gsem2, gsem3, osem0, osem1):
    wid = lax.axis_index("s") * NC + lax.axis_index("c")
    base = wid * QPW
    gsems = (gsem0, gsem1, gsem2, gsem3)
    osems = (osem0, osem1)

    def load_query(qpar, qi):
        pltpu.sync_copy(nidx_hbm.at[base + qi], idx_v.at[qpar])
        pltpu.sync_copy(qn_hbm.at[base + qi], qn_v.at[qpar])

    def fire(slot, qpar, ck):
        pltpu.async_copy(
            table_hbm.at[idx_v.at[qpar, ck]], rows_v.at[slot], gsems[slot])

    def wait_gather(slot, qpar, ck):
        pltpu.make_async_copy(
            table_hbm.at[idx_v.at[qpar, ck]], rows_v.at[slot],
            gsems[slot]).wait()

    lane = lax.broadcasted_iota(jnp.int32, (16,), 0)
    lane15 = jnp.full((16,), 15, jnp.int32)

    def compute(slot, qpar, ck):
        qvecs = [qn_v[qpar, pl.ds(t * 16, 16)] for t in range(DIM // 16)]

        @pl.loop(0, IDX_CHUNK, step=16)
        def _j_loop(j0):
            res = jnp.zeros((16,), jnp.float32)
            for l in range(16):
                j = j0 + l
                acc = rows_v[slot, j, pl.ds(0, 16)] * qvecs[0]
                for t in range(1, DIM // 16):
                    acc = acc + rows_v[slot, j, pl.ds(t * 16, 16)] * qvecs[t]
                tot = jnp.cumsum(acc).at[lane15].get(mode="promise_in_bounds")
                res = jnp.where(lane == l, tot, res)
            out_v[qpar, pl.ds(ck * IDX_CHUNK + j0, 16)] = res

    load_query(0, 0)
    fire(0, 0, 0)
    fire(1, 0, 1)

    @pl.loop(0, NCH, step=2 * NCHUNK)
    def _loop(h0):
        i2 = h0 // NCHUNK
        for b in range(2 * NCHUNK):
            slot = b % NSLOT
            qpar = (b // NCHUNK) % 2
            ck = b % NCHUNK
            qi = i2 + (b // NCHUNK)
            nb = b + 2
            nqpar = (nb // NCHUNK) % 2
            nck = nb % NCHUNK

            @pl.when(i2 + (nb // NCHUNK) < QPW)
            def _prefetch():
                if nck == 0:
                    load_query(nqpar, qi + 1)
                fire((slot + 2) % NSLOT, nqpar, nck)

            wait_gather(slot, qpar, ck)
            if ck == 0:
                @pl.when(qi >= 2)
                def _drain_out():
                    pltpu.make_async_copy(
                        out_v.at[qpar], out_hbm.at[base + qi],
                        osems[qpar]).wait()
            compute(slot, qpar, ck)
            if ck == NCHUNK - 1:
                pltpu.async_copy(
                    out_v.at[qpar], out_hbm.at[base + qi], osems[qpar])

    for p in range(2):
        pltpu.make_async_copy(
            out_v.at[p], out_hbm.at[base], osems[p]).wait()


def _lneg_sc(table_new, qn, nidx3):
    mesh = plsc.VectorSubcoreMesh(core_axis_name="c", subcore_axis_name="s")
    cp = pltpu.CompilerParams()
    if "needs_layout_passes" in pltpu.CompilerParams.__dataclass_fields__:
        cp = dataclasses.replace(cp, needs_layout_passes=False)
    kern = pl.kernel(
        _lneg_sc_body,
        out_type=jax.ShapeDtypeStruct((B, N_NEG), jnp.float32),
        mesh=mesh,
        compiler_params=cp,
        scratch_types=[
            pltpu.VMEM((2, NCHUNK, IDX_CHUNK), jnp.int32),
            pltpu.VMEM((NSLOT, IDX_CHUNK, DIM), jnp.float32),
            pltpu.VMEM((2, DIM), jnp.float32),
            pltpu.VMEM((2, N_NEG), jnp.float32),
            pltpu.SemaphoreType.DMA,
            pltpu.SemaphoreType.DMA,
            pltpu.SemaphoreType.DMA,
            pltpu.SemaphoreType.DMA,
            pltpu.SemaphoreType.DMA,
            pltpu.SemaphoreType.DMA,
        ],
    )
    return kern(table_new, qn, nidx3)


def _epilogue_body(lpos_ref, lneg_ref, out_ref):
    inv_t = jnp.float32(1.0 / T)
    out_ref[...] = jnp.concatenate(
        [lpos_ref[...] * inv_t, lneg_ref[...] * inv_t], axis=1)


def _epilogue(lpos, lneg):
    return pl.pallas_call(
        _epilogue_body,
        out_shape=jax.ShapeDtypeStruct((B, 1 + N_NEG), jnp.float32),
    )(lpos, lneg)


def kernel(q, k, queue, write_idx, neg_idx):
    perm = jnp.argsort(write_idx, stable=True)
    ws = write_idx[perm]
    jstar = jnp.searchsorted(ws, ws, side="right") - 1
    pw = perm[jstar].astype(jnp.int32)
    starts = jnp.searchsorted(
        ws, jnp.arange(NBLK + 1, dtype=jnp.int32) * BK).astype(jnp.int32)
    ws = ws.astype(jnp.int32)

    nidx3 = neg_idx.reshape(B, NCHUNK, IDX_CHUNK)

    qn, kn, lpos = _prologue(q, k)

    table0 = jnp.transpose(queue)
    table_new = _table_scatter(table0, kn, ws, pw, starts)
    lneg = _lneg_sc(table_new, qn, nidx3)
    queue_new = _untranspose(table_new)
    logits = _epilogue(lpos, lneg)
    labels = jnp.zeros((B,), dtype=jnp.int32)
    return logits, queue_new, labels


# speedup vs baseline: 2.1024x; 1.4758x over previous
import dataclasses
import functools

import jax
import jax.numpy as jnp
from jax import lax
from jax.experimental import pallas as pl
from jax.experimental.pallas import tpu as pltpu
from jax.experimental.pallas import tpu_sc as plsc

DIM = 128
K = 262144
B = 4096
N_NEG = 512
T = 0.09

NC = 2
NS = 16
NW = NC * NS
QPW = B // NW
BK = 4096
NBLK = K // BK
IDX_CHUNK = 128
NCHUNK = N_NEG // IDX_CHUNK


def _prologue_body(q_ref, k_ref, qn_ref, kn_ref, lpos_ref):
    qv = q_ref[...]
    kv = k_ref[...]
    qn = qv / jnp.maximum(jnp.sqrt(jnp.sum(qv * qv, axis=1, keepdims=True)), 1e-12)
    kn = kv / jnp.maximum(jnp.sqrt(jnp.sum(kv * kv, axis=1, keepdims=True)), 1e-12)
    qn_ref[...] = qn
    kn_ref[...] = kn
    lpos_ref[...] = jnp.sum(qn * kn, axis=1, keepdims=True)


def _prologue(q, k):
    return pl.pallas_call(
        _prologue_body,
        out_shape=(
            jax.ShapeDtypeStruct((B, DIM), jnp.float32),
            jax.ShapeDtypeStruct((B, DIM), jnp.float32),
            jax.ShapeDtypeStruct((B, 1), jnp.float32),
        ),
    )(q, k)


def _qupd_body(ws_ref, pw_ref, starts_ref, q_ref, kn_ref, tbl_ref, qnew_ref):
    i = pl.program_id(0)
    tbl_ref[...] = jnp.transpose(q_ref[...])
    s = starts_ref[i]
    e = starts_ref[i + 1]

    def wr(j, carry):
        row = ws_ref[j] - i * BK
        src = pw_ref[j]
        tbl_ref[pl.ds(row, 1), :] = kn_ref[pl.ds(src, 1), :]
        return carry

    lax.fori_loop(s, e, wr, 0)
    qnew_ref[...] = jnp.transpose(tbl_ref[...])


def _queue_update(queue, kn, ws, pw, starts):
    return pl.pallas_call(
        _qupd_body,
        grid=(NBLK,),
        in_specs=[
            pl.BlockSpec(memory_space=pltpu.SMEM),
            pl.BlockSpec(memory_space=pltpu.SMEM),
            pl.BlockSpec(memory_space=pltpu.SMEM),
            pl.BlockSpec((DIM, BK), lambda i: (0, i)),
            pl.BlockSpec((B, DIM), lambda i: (0, 0)),
        ],
        out_specs=(
            pl.BlockSpec((BK, DIM), lambda i: (i, 0)),
            pl.BlockSpec((DIM, BK), lambda i: (0, i)),
        ),
        out_shape=(
            jax.ShapeDtypeStruct((K, DIM), jnp.float32),
            jax.ShapeDtypeStruct((DIM, K), jnp.float32),
        ),
    )(ws, pw, starts, queue, kn)


NSLOT = 4
NCH = QPW * NCHUNK


def _lneg_sc_body(table_hbm, qn_hbm, nidx_hbm, out_hbm,
                  idx_v, rows_v, qn_v, out_v,
                  gsem0, gsem1, gsem2, gsem3, osem0, osem1, isem0, isem1):
    wid = lax.axis_index("s") * NC + lax.axis_index("c")
    base = wid * QPW
    gsems = (gsem0, gsem1, gsem2, gsem3)
    osems = (osem0, osem1)
    isems = (isem0, isem1)

    def load_query(qpar, qi):
        pltpu.async_copy(nidx_hbm.at[base + qi], idx_v.at[qpar], isems[qpar])
        pltpu.async_copy(qn_hbm.at[base + qi], qn_v.at[qpar], isems[qpar])

    def wait_query(qpar, qi):
        pltpu.make_async_copy(
            nidx_hbm.at[base + qi], idx_v.at[qpar], isems[qpar]).wait()
        pltpu.make_async_copy(
            qn_hbm.at[base + qi], qn_v.at[qpar], isems[qpar]).wait()

    def fire(slot, qpar, ck):
        pltpu.async_copy(
            table_hbm.at[idx_v.at[qpar, ck]], rows_v.at[slot], gsems[slot])

    def wait_gather(slot, qpar, ck):
        pltpu.make_async_copy(
            table_hbm.at[idx_v.at[qpar, ck]], rows_v.at[slot],
            gsems[slot]).wait()

    lane = lax.broadcasted_iota(jnp.int32, (16,), 0)
    lane15 = jnp.full((16,), 15, jnp.int32)

    def compute(slot, qpar, ck):
        qvecs = [qn_v[qpar, pl.ds(t * 16, 16)] for t in range(DIM // 16)]

        @pl.loop(0, IDX_CHUNK, step=16)
        def _j_loop(j0):
            res = jnp.zeros((16,), jnp.float32)
            for l in range(16):
                j = j0 + l
                acc = rows_v[slot, j, pl.ds(0, 16)] * qvecs[0]
                for t in range(1, DIM // 16):
                    acc = acc + rows_v[slot, j, pl.ds(t * 16, 16)] * qvecs[t]
                tot = jnp.cumsum(acc).at[lane15].get(mode="promise_in_bounds")
                res = jnp.where(lane == l, tot, res)
            out_v[qpar, pl.ds(ck * IDX_CHUNK + j0, 16)] = res

    load_query(0, 0)
    wait_query(0, 0)
    fire(0, 0, 0)
    fire(1, 0, 1)

    @pl.loop(0, NCH, step=2 * NCHUNK)
    def _loop(h0):
        i2 = h0 // NCHUNK
        for b in range(2 * NCHUNK):
            slot = b % NSLOT
            qpar = (b // NCHUNK) % 2
            ck = b % NCHUNK
            qi = i2 + (b // NCHUNK)

            if b == 0:
                load_query(1, i2 + 1)
            if b == NCHUNK:
                @pl.when(i2 + 2 < QPW)
                def _stage_next():
                    load_query(0, i2 + 2)

            nb = b + 2
            nqpar = (nb // NCHUNK) % 2
            nck = nb % NCHUNK

            @pl.when(i2 + (nb // NCHUNK) < QPW)
            def _prefetch():
                if nck == 0:
                    wait_query(nqpar, qi + 1)
                fire((slot + 2) % NSLOT, nqpar, nck)

            wait_gather(slot, qpar, ck)
            if ck == 0:
                @pl.when(qi >= 2)
                def _drain_out():
                    pltpu.make_async_copy(
                        out_v.at[qpar], out_hbm.at[base + qi],
                        osems[qpar]).wait()
            compute(slot, qpar, ck)
            if ck == NCHUNK - 1:
                pltpu.async_copy(
                    out_v.at[qpar], out_hbm.at[base + qi], osems[qpar])

    for p in range(2):
        pltpu.make_async_copy(
            out_v.at[p], out_hbm.at[base], osems[p]).wait()


def _lneg_sc(table_new, qn, nidx3):
    mesh = plsc.VectorSubcoreMesh(core_axis_name="c", subcore_axis_name="s")
    cp = pltpu.CompilerParams()
    if "needs_layout_passes" in pltpu.CompilerParams.__dataclass_fields__:
        cp = dataclasses.replace(cp, needs_layout_passes=False)
    kern = pl.kernel(
        _lneg_sc_body,
        out_type=jax.ShapeDtypeStruct((B, N_NEG), jnp.float32),
        mesh=mesh,
        compiler_params=cp,
        scratch_types=[
            pltpu.VMEM((2, NCHUNK, IDX_CHUNK), jnp.int32),
            pltpu.VMEM((NSLOT, IDX_CHUNK, DIM), jnp.float32),
            pltpu.VMEM((2, DIM), jnp.float32),
            pltpu.VMEM((2, N_NEG), jnp.float32),
            pltpu.SemaphoreType.DMA,
            pltpu.SemaphoreType.DMA,
            pltpu.SemaphoreType.DMA,
            pltpu.SemaphoreType.DMA,
            pltpu.SemaphoreType.DMA,
            pltpu.SemaphoreType.DMA,
            pltpu.SemaphoreType.DMA,
            pltpu.SemaphoreType.DMA,
        ],
    )
    return kern(table_new, qn, nidx3)


def _epilogue_body(lpos_ref, lneg_ref, out_ref):
    inv_t = jnp.float32(1.0 / T)
    out_ref[...] = jnp.concatenate(
        [lpos_ref[...] * inv_t, lneg_ref[...] * inv_t], axis=1)


def _epilogue(lpos, lneg):
    return pl.pallas_call(
        _epilogue_body,
        out_shape=jax.ShapeDtypeStruct((B, 1 + N_NEG), jnp.float32),
    )(lpos, lneg)


def kernel(q, k, queue, write_idx, neg_idx):
    barange = jnp.arange(B, dtype=jnp.int32)
    skey = jnp.sort(write_idx * B + barange)
    ws = (skey // B).astype(jnp.int32)
    pb = (skey % B).astype(jnp.int32)
    is_first = jnp.concatenate(
        [jnp.ones((1,), jnp.bool_), ws[1:] != ws[:-1]])
    run_id = jnp.cumsum(is_first.astype(jnp.int32))
    key2 = (B + 1 - run_id) * B + pb
    rcm = jnp.flip(lax.cummax(jnp.flip(key2), axis=0))
    pw = (rcm % B).astype(jnp.int32)
    bid = ws // BK
    counts = jnp.sum(
        (bid[None, :] == jnp.arange(NBLK, dtype=jnp.int32)[:, None])
        .astype(jnp.int32), axis=1)
    starts = jnp.concatenate(
        [jnp.zeros((1,), jnp.int32), jnp.cumsum(counts)]).astype(jnp.int32)

    nidx3 = neg_idx.reshape(B, NCHUNK, IDX_CHUNK)

    qn, kn, lpos = _prologue(q, k)
    table_new, queue_new = _queue_update(queue, kn, ws, pw, starts)
    lneg = _lneg_sc(table_new, qn, nidx3)
    logits = _epilogue(lpos, lneg)
    labels = jnp.zeros((B,), dtype=jnp.int32)
    return logits, queue_new, labels
